# Initial kernel scaffold; baseline (speedup 1.0000x reference)
import jax
import jax.numpy as jnp
from jax.experimental import pallas as pl


def kernel(feat, mem_k, mem_v, mem_c, mem_attn, g_q, b_q, g_k, b_k, g_v, b_v):
    raise NotImplementedError("write your pallas kernel here")


# speedup vs baseline: 1.8463x; 1.8463x over previous
import jax
import jax.numpy as jnp
from jax.experimental import pallas as pl
from jax.experimental.pallas import tpu as pltpu

LN_EPS = 1e-5
ATTN_THRESH = 0.0005


def _ln(x, g, b):
    mu = jnp.mean(x, axis=-1, keepdims=True)
    var = jnp.mean((x - mu) ** 2, axis=-1, keepdims=True)
    return (x - mu) / jnp.sqrt(var + LN_EPS) * g + b


def _body(feat_ref, k_ref, v_ref, c_ref, gq_ref, bq_ref, gk_ref, bk_ref,
          gv_ref, bv_ref, o_ref, qt_ref, st_ref, m_ref, z_ref, d_ref,
          *, nxb, xb, c_dim):
    ph = pl.program_id(1)
    j = pl.program_id(2)
    inv_sqrt_c = 1.0 / (c_dim ** 0.5)

    @pl.when((ph == 0) & (j == 0))
    def _():
        q = _ln(feat_ref[0], gq_ref[...], bq_ref[...])
        qt_ref[...] = q.T
        m_ref[...] = jnp.full_like(m_ref, -jnp.inf)
        z_ref[...] = jnp.zeros_like(z_ref)
        d_ref[...] = jnp.zeros_like(d_ref)
        o_ref[...] = jnp.zeros_like(o_ref)

    @pl.when(ph == 0)
    def _():
        k = _ln(k_ref[0], gk_ref[...], bk_ref[...])
        s = jax.lax.dot(k, qt_ref[...],
                        preferred_element_type=jnp.float32)
        s = s * (c_ref[0] * inv_sqrt_c)
        st_ref[pl.ds(j * xb, xb), :] = s.astype(jnp.bfloat16)
        m_old = m_ref[...]
        m_new = jnp.maximum(m_old, jnp.max(s, axis=0, keepdims=True))
        z_ref[...] = (z_ref[...] * jnp.exp(m_old - m_new)
                      + jnp.sum(jnp.exp(s - m_new), axis=0, keepdims=True))
        m_ref[...] = m_new

    @pl.when(ph == 1)
    def _():
        s = st_ref[pl.ds(j * xb, xb), :].astype(jnp.float32)
        e = jnp.exp(s - m_ref[...])
        e = jnp.where(e < z_ref[...] * ATTN_THRESH, 0.0, e)
        d_ref[...] += jnp.sum(e, axis=0, keepdims=True)
        v = _ln(v_ref[0], gv_ref[...], bv_ref[...])
        o_ref[0] += jax.lax.dot_general(
            e, v, (((0,), (0,)), ((), ())),
            preferred_element_type=jnp.float32)

    @pl.when((ph == 1) & (j == nxb - 1))
    def _():
        den = d_ref[...].T
        o_ref[0] = o_ref[0] / den + feat_ref[0]


def kernel(feat, mem_k, mem_v, mem_c, mem_attn, g_q, b_q, g_k, b_k, g_v, b_v):
    del mem_attn
    B, P, C = feat.shape
    X = mem_k.shape[1]
    XB = min(512, X)
    NXB = X // XB

    g_q2, b_q2 = g_q.reshape(1, C), b_q.reshape(1, C)
    g_k2, b_k2 = g_k.reshape(1, C), b_k.reshape(1, C)
    g_v2, b_v2 = g_v.reshape(1, C), b_v.reshape(1, C)

    def w_spec():
        return pl.BlockSpec((1, C), lambda b, ph, j: (0, 0))

    import functools
    body = functools.partial(_body, nxb=NXB, xb=XB, c_dim=C)

    return pl.pallas_call(
        body,
        out_shape=jax.ShapeDtypeStruct((B, P, C), jnp.float32),
        grid=(B, 2, NXB),
        in_specs=[
            pl.BlockSpec((1, P, C), lambda b, ph, j: (b, 0, 0)),
            pl.BlockSpec((1, XB, C),
                         lambda b, ph, j: (b, jnp.where(ph == 0, j, 0), 0)),
            pl.BlockSpec((1, XB, C),
                         lambda b, ph, j: (b, jnp.where(ph == 0, 0, j), 0)),
            pl.BlockSpec((1, XB, 1),
                         lambda b, ph, j: (b, jnp.where(ph == 0, j, 0), 0)),
            w_spec(), w_spec(), w_spec(), w_spec(), w_spec(), w_spec(),
        ],
        out_specs=pl.BlockSpec((1, P, C), lambda b, ph, j: (b, 0, 0)),
        scratch_shapes=[
            pltpu.VMEM((C, P), jnp.float32),
            pltpu.VMEM((X, P), jnp.bfloat16),
            pltpu.VMEM((1, P), jnp.float32),
            pltpu.VMEM((1, P), jnp.float32),
            pltpu.VMEM((1, P), jnp.float32),
        ],
        compiler_params=pltpu.CompilerParams(
            dimension_semantics=("parallel", "arbitrary", "arbitrary"),
            vmem_limit_bytes=56 * 1024 * 1024,
        ),
        name="spann3r_memory_read",
    )(feat, mem_k, mem_v, mem_c, g_q2, b_q2, g_k2, b_k2, g_v2, b_v2)


# speedup vs baseline: 2.0553x; 1.1132x over previous
import jax
import jax.numpy as jnp
from jax.experimental import pallas as pl
from jax.experimental.pallas import tpu as pltpu

LN_EPS = 1e-5
ATTN_THRESH = 0.0005


def _ln(x, g, b):
    mu = jnp.mean(x, axis=-1, keepdims=True)
    var = jnp.mean((x - mu) ** 2, axis=-1, keepdims=True)
    return (x - mu) / jnp.sqrt(var + LN_EPS) * g + b


def _body(feat_ref, k_ref, v_ref, c_ref, gq_ref, bq_ref, gk_ref, bk_ref,
          gv_ref, bv_ref, o_ref, qt_ref, st_ref, m_ref, z_ref, d_ref,
          *, nxb, xb, c_dim):
    ph = pl.program_id(1)
    j = pl.program_id(2)
    inv_sqrt_c = 1.0 / (c_dim ** 0.5)

    @pl.when((ph == 0) & (j == 0))
    def _():
        q = _ln(feat_ref[0], gq_ref[...], bq_ref[...])
        qt_ref[...] = q.T
        m_ref[...] = jnp.full_like(m_ref, -jnp.inf)
        z_ref[...] = jnp.zeros_like(z_ref)
        d_ref[...] = jnp.zeros_like(d_ref)
        o_ref[...] = jnp.zeros_like(o_ref)

    @pl.when(ph == 0)
    def _():
        k = _ln(k_ref[0], gk_ref[...], bk_ref[...])
        s = jax.lax.dot(k, qt_ref[...],
                        preferred_element_type=jnp.float32)
        s = s * (c_ref[0].T * inv_sqrt_c)
        st_ref[pl.ds(j * xb, xb), :] = s.astype(jnp.bfloat16)
        m_old = m_ref[...]
        m_new = jnp.maximum(m_old, jnp.max(s, axis=0, keepdims=True))
        z_ref[...] = (z_ref[...] * jnp.exp(m_old - m_new)
                      + jnp.sum(jnp.exp(s - m_new), axis=0, keepdims=True))
        m_ref[...] = m_new

    @pl.when(ph == 1)
    def _():
        s = st_ref[pl.ds(j * xb, xb), :].astype(jnp.float32)
        e = jnp.exp(s - m_ref[...])
        e = jnp.where(e < z_ref[...] * ATTN_THRESH, 0.0, e)
        d_ref[...] += jnp.sum(e, axis=0, keepdims=True)
        v = _ln(v_ref[0], gv_ref[...], bv_ref[...])
        o_ref[0] += jax.lax.dot_general(
            e, v, (((0,), (0,)), ((), ())),
            preferred_element_type=jnp.float32)

    @pl.when((ph == 1) & (j == nxb - 1))
    def _():
        den = d_ref[...].T
        o_ref[0] = o_ref[0] / den + feat_ref[0]


def kernel(feat, mem_k, mem_v, mem_c, mem_attn, g_q, b_q, g_k, b_k, g_v, b_v):
    del mem_attn
    B, P, C = feat.shape
    X = mem_k.shape[1]
    XB = min(1024, X)
    NXB = X // XB

    g_q2, b_q2 = g_q.reshape(1, C), b_q.reshape(1, C)
    g_k2, b_k2 = g_k.reshape(1, C), b_k.reshape(1, C)
    g_v2, b_v2 = g_v.reshape(1, C), b_v.reshape(1, C)

    def w_spec():
        return pl.BlockSpec((1, C), lambda b, ph, j: (0, 0))

    import functools
    body = functools.partial(_body, nxb=NXB, xb=XB, c_dim=C)

    return pl.pallas_call(
        body,
        out_shape=jax.ShapeDtypeStruct((B, P, C), jnp.float32),
        grid=(B, 2, NXB),
        in_specs=[
            pl.BlockSpec((1, P, C), lambda b, ph, j: (b, 0, 0)),
            pl.BlockSpec((1, XB, C),
                         lambda b, ph, j: (b, jnp.where(ph == 0, j, 0), 0)),
            pl.BlockSpec((1, XB, C),
                         lambda b, ph, j: (b, jnp.where(ph == 0, 0, j), 0)),
            pl.BlockSpec((1, 1, XB),
                         lambda b, ph, j: (b, 0, jnp.where(ph == 0, j, 0))),
            w_spec(), w_spec(), w_spec(), w_spec(), w_spec(), w_spec(),
        ],
        out_specs=pl.BlockSpec((1, P, C), lambda b, ph, j: (b, 0, 0)),
        scratch_shapes=[
            pltpu.VMEM((C, P), jnp.float32),
            pltpu.VMEM((X, P), jnp.bfloat16),
            pltpu.VMEM((1, P), jnp.float32),
            pltpu.VMEM((1, P), jnp.float32),
            pltpu.VMEM((1, P), jnp.float32),
        ],
        compiler_params=pltpu.CompilerParams(
            dimension_semantics=("parallel", "arbitrary", "arbitrary"),
            vmem_limit_bytes=60000 * 1024,
        ),
        name="spann3r_memory_read",
    )(feat, mem_k, mem_v, mem_c.reshape(B, 1, X), g_q2, b_q2, g_k2, b_k2,
      g_v2, b_v2)


# speedup vs baseline: 2.1950x; 1.0680x over previous
import functools

import jax
import jax.numpy as jnp
from jax.experimental import pallas as pl
from jax.experimental.pallas import tpu as pltpu

LN_EPS = 1e-5
ATTN_THRESH = 0.0005


def _ln(x, g, b):
    mu = jnp.mean(x, axis=-1, keepdims=True)
    var = jnp.mean((x - mu) ** 2, axis=-1, keepdims=True)
    return (x - mu) / jnp.sqrt(var + LN_EPS) * g + b


def _norm(x):
    mu = jnp.mean(x, axis=-1, keepdims=True)
    ms = jnp.mean(x * x, axis=-1, keepdims=True)
    var = ms - mu * mu
    return (x - mu) * jax.lax.rsqrt(var + LN_EPS)


def _body(feat_ref, k_ref, v_ref, c_ref, gq_ref, bq_ref, gk_ref, bk_ref,
          gv_ref, bv_ref, o_ref, qgk_ref, et_ref, w_ref, m_ref, mh_ref,
          z_ref, d_ref, *, nxb, xb, c_dim):
    ph = pl.program_id(1)
    j = pl.program_id(2)
    inv_sqrt_c = 1.0 / (c_dim ** 0.5)

    @pl.when((ph == 0) & (j == 0))
    def _():
        q = _ln(feat_ref[0], gq_ref[...], bq_ref[...])
        qt = q.T
        qgk_ref[...] = qt * gk_ref[...].T
        w_ref[...] = jnp.sum(qt * bk_ref[...].T, axis=0,
                             keepdims=True)
        m_ref[...] = jnp.full_like(m_ref, -jnp.inf)
        z_ref[...] = jnp.zeros_like(z_ref)
        d_ref[...] = jnp.zeros_like(d_ref)
        o_ref[...] = jnp.zeros_like(o_ref)

    @pl.when(ph == 0)
    def _():
        nk = _norm(k_ref[0])
        t = jax.lax.dot(nk, qgk_ref[...],
                        preferred_element_type=jnp.float32)
        s = (t + w_ref[...]) * (c_ref[0].T * inv_sqrt_c)
        m_old = m_ref[...]
        m_new = jnp.maximum(m_old, jnp.max(s, axis=0, keepdims=True))
        e = jnp.exp(s - m_new)
        et_ref[pl.ds(j * xb, xb), :] = e.astype(jnp.bfloat16)
        mh_ref[pl.ds(j, 1), :] = m_new
        z_ref[...] = (z_ref[...] * jnp.exp(m_old - m_new)
                      + jnp.sum(e, axis=0, keepdims=True))
        m_ref[...] = m_new

    @pl.when(ph == 1)
    def _():
        e = et_ref[pl.ds(j * xb, xb), :].astype(jnp.float32)
        r = jnp.exp(mh_ref[pl.ds(j, 1), :] - m_ref[...])
        f = jnp.where(e * r < z_ref[...] * ATTN_THRESH, 0.0, e * r)
        d_ref[...] += jnp.sum(f, axis=0, keepdims=True)
        nv = _norm(v_ref[0])
        o_ref[0] += jax.lax.dot_general(
            f, nv, (((0,), (0,)), ((), ())),
            preferred_element_type=jnp.float32)

    @pl.when((ph == 1) & (j == nxb - 1))
    def _():
        den = d_ref[...].T
        o_ref[0] = (o_ref[0] * gv_ref[...]) / den + bv_ref[...] + feat_ref[0]


def kernel(feat, mem_k, mem_v, mem_c, mem_attn, g_q, b_q, g_k, b_k, g_v, b_v):
    del mem_attn
    B, P, C = feat.shape
    X = mem_k.shape[1]
    XB = min(1024, X)
    NXB = X // XB

    g_q2, b_q2 = g_q.reshape(1, C), b_q.reshape(1, C)
    g_k2, b_k2 = g_k.reshape(1, C), b_k.reshape(1, C)
    g_v2, b_v2 = g_v.reshape(1, C), b_v.reshape(1, C)

    def w_spec():
        return pl.BlockSpec((1, C), lambda b, ph, j: (0, 0))

    body = functools.partial(_body, nxb=NXB, xb=XB, c_dim=C)

    return pl.pallas_call(
        body,
        out_shape=jax.ShapeDtypeStruct((B, P, C), jnp.float32),
        grid=(B, 2, NXB),
        in_specs=[
            pl.BlockSpec((1, P, C), lambda b, ph, j: (b, 0, 0)),
            pl.BlockSpec((1, XB, C),
                         lambda b, ph, j: (b, jnp.where(ph == 0, j, 0), 0)),
            pl.BlockSpec((1, XB, C),
                         lambda b, ph, j: (b, jnp.where(ph == 0, 0, j), 0)),
            pl.BlockSpec((1, 1, XB),
                         lambda b, ph, j: (b, 0, jnp.where(ph == 0, j, 0))),
            w_spec(), w_spec(), w_spec(), w_spec(), w_spec(), w_spec(),
        ],
        out_specs=pl.BlockSpec((1, P, C), lambda b, ph, j: (b, 0, 0)),
        scratch_shapes=[
            pltpu.VMEM((C, P), jnp.float32),
            pltpu.VMEM((X, P), jnp.bfloat16),
            pltpu.VMEM((1, P), jnp.float32),
            pltpu.VMEM((1, P), jnp.float32),
            pltpu.VMEM((NXB, P), jnp.float32),
            pltpu.VMEM((1, P), jnp.float32),
            pltpu.VMEM((1, P), jnp.float32),
        ],
        compiler_params=pltpu.CompilerParams(
            dimension_semantics=("parallel", "arbitrary", "arbitrary"),
            vmem_limit_bytes=60000 * 1024,
        ),
        name="spann3r_memory_read",
    )(feat, mem_k, mem_v, mem_c.reshape(B, 1, X), g_q2, b_q2, g_k2, b_k2,
      g_v2, b_v2)


# speedup vs baseline: 2.1962x; 1.0006x over previous
import functools

import jax
import jax.numpy as jnp
from jax.experimental import pallas as pl
from jax.experimental.pallas import tpu as pltpu

LN_EPS = 1e-5
ATTN_THRESH = 0.0005


def _norm(x):
    mu = jnp.mean(x, axis=-1, keepdims=True)
    ms = jnp.mean(x * x, axis=-1, keepdims=True)
    var = ms - mu * mu
    return (x - mu) * jax.lax.rsqrt(var + LN_EPS)


def _body(feat_ref, k_ref, v_ref, c_ref, gq_ref, bq_ref, gk_ref, bk_ref,
          gv_ref, bv_ref, o_ref, qgk_ref, et_ref, w_ref, m_ref, mh_ref,
          z_ref, d_ref, *, nxb, xb, c_dim):
    ph = pl.program_id(1)
    j = pl.program_id(2)
    inv_sqrt_c = 1.0 / (c_dim ** 0.5)

    @pl.when((ph == 0) & (j == 0))
    def _():
        ggk = (gq_ref[...] * gk_ref[...]).T
        bgk = (bq_ref[...] * gk_ref[...]).T
        gbk = (gq_ref[...] * bk_ref[...]).T
        qgk_ref[...] = _norm(feat_ref[0]).T
        w_ref[...] = (jnp.sum(qgk_ref[...] * gbk, axis=0, keepdims=True)
                      + jnp.sum(bq_ref[...] * bk_ref[...]))
        qgk_ref[...] = qgk_ref[...] * ggk + bgk
        m_ref[...] = jnp.full_like(m_ref, -jnp.inf)
        z_ref[...] = jnp.zeros_like(z_ref)
        d_ref[...] = jnp.zeros_like(d_ref)
        o_ref[...] = jnp.zeros_like(o_ref)

    @pl.when(ph == 0)
    def _():
        nk = _norm(k_ref[0])
        t = jax.lax.dot(nk, qgk_ref[...],
                        preferred_element_type=jnp.float32)
        s = (t + w_ref[...]) * (c_ref[0].T * inv_sqrt_c)
        m_old = m_ref[...]
        m_new = jnp.maximum(m_old, jnp.max(s, axis=0, keepdims=True))
        e = jnp.exp(s - m_new)
        et_ref[pl.ds(j * xb, xb), :] = e.astype(jnp.bfloat16)
        mh_ref[pl.ds(j, 1), :] = m_new
        z_ref[...] = (z_ref[...] * jnp.exp(m_old - m_new)
                      + jnp.sum(e, axis=0, keepdims=True))
        m_ref[...] = m_new

    @pl.when(ph == 1)
    def _():
        e = et_ref[pl.ds(j * xb, xb), :].astype(jnp.float32)
        r = jnp.exp(mh_ref[pl.ds(j, 1), :] - m_ref[...])
        f = jnp.where(e * r < z_ref[...] * ATTN_THRESH, 0.0, e * r)
        d_ref[...] += jnp.sum(f, axis=0, keepdims=True)
        nv = _norm(v_ref[0])
        o_ref[0] += jax.lax.dot_general(
            f, nv, (((0,), (0,)), ((), ())),
            preferred_element_type=jnp.float32)

    @pl.when((ph == 1) & (j == nxb - 1))
    def _():
        den = d_ref[...].T
        o_ref[0] = (o_ref[0] * gv_ref[...]) / den + bv_ref[...] + feat_ref[0]


def kernel(feat, mem_k, mem_v, mem_c, mem_attn, g_q, b_q, g_k, b_k, g_v, b_v):
    del mem_attn
    B, P, C = feat.shape
    X = mem_k.shape[1]
    XB = min(1024, X)
    NXB = X // XB

    g_q2, b_q2 = g_q.reshape(1, C), b_q.reshape(1, C)
    g_k2, b_k2 = g_k.reshape(1, C), b_k.reshape(1, C)
    g_v2, b_v2 = g_v.reshape(1, C), b_v.reshape(1, C)

    def w_spec():
        return pl.BlockSpec((1, C), lambda b, ph, j: (0, 0))

    body = functools.partial(_body, nxb=NXB, xb=XB, c_dim=C)

    return pl.pallas_call(
        body,
        out_shape=jax.ShapeDtypeStruct((B, P, C), jnp.float32),
        grid=(B, 2, NXB),
        in_specs=[
            pl.BlockSpec((1, P, C), lambda b, ph, j: (b, 0, 0)),
            pl.BlockSpec((1, XB, C),
                         lambda b, ph, j: (b, jnp.where(ph == 0, j, 0), 0)),
            pl.BlockSpec((1, XB, C),
                         lambda b, ph, j: (b, jnp.where(ph == 0, 0, j), 0)),
            pl.BlockSpec((1, 1, XB),
                         lambda b, ph, j: (b, 0, jnp.where(ph == 0, j, 0))),
            w_spec(), w_spec(), w_spec(), w_spec(), w_spec(), w_spec(),
        ],
        out_specs=pl.BlockSpec((1, P, C), lambda b, ph, j: (b, 0, 0)),
        scratch_shapes=[
            pltpu.VMEM((C, P), jnp.float32),
            pltpu.VMEM((X, P), jnp.bfloat16),
            pltpu.VMEM((1, P), jnp.float32),
            pltpu.VMEM((1, P), jnp.float32),
            pltpu.VMEM((NXB, P), jnp.float32),
            pltpu.VMEM((1, P), jnp.float32),
            pltpu.VMEM((1, P), jnp.float32),
        ],
        compiler_params=pltpu.CompilerParams(
            dimension_semantics=("parallel", "arbitrary", "arbitrary"),
            vmem_limit_bytes=60000 * 1024,
        ),
        name="spann3r_memory_read",
    )(feat, mem_k, mem_v, mem_c.reshape(B, 1, X), g_q2, b_q2, g_k2, b_k2,
      g_v2, b_v2)
